# baseline (speedup 1.0000x reference)
import functools

import jax.experimental.shard_map

import numpy as np

import jax
import jax.numpy as jnp
from jax.experimental import pallas as pl
from jax.experimental.pallas import tpu as pltpu

_KNOTS = 16
_BOUND = 5.0
_LOWER = 3
_TN = 4096


def _build_tmat():
    t = np.zeros((54, 96), np.float32)
    for d in range(3):
        for k in range(16):
            t[16 * d + k, 16 * d : 16 * d + k + 1] = 1.0
        t[48 + d, 16 * d : 16 * d + 16] = 1.0
        t[51 + d, 48 + 16 * d : 48 + 16 * d + 16] = 1.0
    return t


def _spline_body(xc_ref, w1_ref, b1_ref, w2_ref, b2_ref, w3_ref,
                 b3_ref, t_ref, y_ref, ld_ref):
    xc = xc_ref[...]
    h = xc[3:11, :]
    a = jnp.dot(w1_ref[...], h, preferred_element_type=jnp.float32)
    a = jnp.maximum(a + b1_ref[...], 0.0)
    a = jnp.dot(w2_ref[...], a, preferred_element_type=jnp.float32)
    a = jnp.maximum(a + b2_ref[...], 0.0)
    p = jnp.dot(w3_ref[...], a, preferred_element_type=jnp.float32)
    p = p + b3_ref[...]

    lo = xc[0:3, :]
    tn = lo.shape[1]

    epieces = []
    for g in range(6):
        lg = p[16 * g : 16 * g + 16, :]
        epieces.append(jnp.exp(lg - jnp.max(lg, axis=0, keepdims=True)))
    e_all = jnp.concatenate(epieces, axis=0)
    dlog = p[96:144, :]
    d_all = jnp.maximum(dlog, 0.0) + jnp.log1p(jnp.exp(-jnp.abs(dlog)))

    c_mat = jnp.dot(t_ref[...], e_all,
                    preferred_element_type=jnp.float32,
                    precision=jax.lax.Precision.HIGHEST)

    oob = (lo <= -_BOUND) | (lo >= _BOUND)
    xm = jnp.where(oob, -_BOUND, lo)
    sw3 = c_mat[48:51, :]
    sh3 = c_mat[51:54, :]
    t3 = (xm + _BOUND) * sw3 * (1.0 / (2.0 * _BOUND))

    kposf = jax.lax.broadcasted_iota(jnp.int32, (_KNOTS, tn), 0).astype(
        jnp.float32)
    idx_pieces = []
    xpieces = []
    for d in range(3):
        cwe = c_mat[16 * d : 16 * d + 16, :]
        td = t3[d : d + 1, :]
        cnt = jnp.sum(jnp.where(td >= cwe, 1.0, 0.0), axis=0, keepdims=True)
        idxd = jnp.clip(cnt, 0.0, 15.0)
        mlt = jnp.where(kposf < idxd, 1.0, 0.0)
        seq = jnp.where(kposf == idxd, 1.0, 0.0)
        sm1 = jnp.where(kposf == idxd - 1.0, 1.0, 0.0)
        ew = e_all[16 * d : 16 * d + 16, :]
        eh = e_all[48 + 16 * d : 48 + 16 * d + 16, :]
        dd = d_all[16 * d : 16 * d + 16, :]
        xpieces.append([
            jnp.sum(ew * seq, axis=0, keepdims=True),
            jnp.sum(eh * seq, axis=0, keepdims=True),
            jnp.sum(ew * mlt, axis=0, keepdims=True),
            jnp.sum(eh * mlt, axis=0, keepdims=True),
            jnp.sum(dd * sm1, axis=0, keepdims=True),
            jnp.sum(dd * seq, axis=0, keepdims=True),
        ])
        idx_pieces.append(idxd)
    idx3 = jnp.concatenate(idx_pieces, axis=0)
    s_mat = jnp.concatenate(
        [xpieces[d][q] for q in range(6) for d in range(3)], axis=0)

    rw3 = (2.0 * _BOUND) / sw3
    rh3 = (2.0 * _BOUND) / sh3
    wk = s_mat[0:3, :] * rw3
    hk = s_mat[3:6, :] * rh3
    xkb = s_mat[6:9, :] * rw3 - _BOUND
    ykb = s_mat[9:12, :] * rh3 - _BOUND
    dkb = jnp.where(idx3 == 0.0, 1.0, s_mat[12:15, :])
    dk1 = jnp.where(idx3 == 15.0, 1.0, s_mat[15:18, :])

    rwk = 1.0 / wk
    sk = hk * rwk
    relx = jnp.clip((xm - xkb) * rwk, 0.0, 1.0)
    omr = 1.0 - relx
    r1 = relx * omr
    den = sk + (dk1 + dkb - 2.0 * sk) * r1
    num = hk * (sk * relx * relx + dkb * r1)
    y3 = ykb + num / den
    ld3 = (2.0 * jnp.log(sk)
           + jnp.log(dk1 * relx * relx + 2.0 * sk * r1 + dkb * omr * omr)
           - 2.0 * jnp.log(den))
    y3 = jnp.where(oob, lo, y3)
    ld3 = jnp.where(oob, 0.0, ld3)

    y_ref[0:3, :] = y3
    y_ref[3:6, :] = h[0:3, :]
    ld_ref[...] = jnp.sum(ld3, axis=0, keepdims=True)


def _coupling(x, c, W1, b1, W2, b2, W3, b3, interpret=False):
    n = x.shape[0]
    g = -(-n // _TN)
    npad = g * _TN - n

    xc = jnp.concatenate([x, c, jnp.zeros((n, 1), jnp.float32)], axis=1)
    xcT = jnp.pad(jnp.transpose(xc), ((0, 0), (0, npad)))

    w1t = jnp.pad(jnp.transpose(W1), ((0, 0), (0, 1)))
    w2t = jnp.transpose(W2)
    w3t = jnp.transpose(W3)
    z1 = jnp.zeros((1, 128), jnp.float32)
    rows = []
    brows = []
    for off, width in ((0, 16), (16, 16), (32, 15)):
        for d in range(_LOWER):
            s = 47 * d + off
            rows.append(w3t[s:s + width])
            brows.append(b3[s:s + width])
            if width == 15:
                rows.append(z1)
                brows.append(jnp.zeros((1,), jnp.float32))
    w3g = jnp.concatenate(rows, axis=0)
    b3g = jnp.concatenate(brows, axis=0).reshape(144, 1)

    tmat = jnp.asarray(_build_tmat())

    np_tot = n + npad
    yT, ldT = pl.pallas_call(
        _spline_body,
        grid=(g,),
        in_specs=[
            pl.BlockSpec((11, _TN), lambda i: (0, i)),
            pl.BlockSpec((128, 8), lambda i: (0, 0)),
            pl.BlockSpec((128, 1), lambda i: (0, 0)),
            pl.BlockSpec((128, 128), lambda i: (0, 0)),
            pl.BlockSpec((128, 1), lambda i: (0, 0)),
            pl.BlockSpec((144, 128), lambda i: (0, 0)),
            pl.BlockSpec((144, 1), lambda i: (0, 0)),
            pl.BlockSpec((54, 96), lambda i: (0, 0)),
        ],
        out_specs=[
            pl.BlockSpec((6, _TN), lambda i: (0, i)),
            pl.BlockSpec((1, _TN), lambda i: (0, i)),
        ],
        out_shape=[
            jax.ShapeDtypeStruct((6, np_tot), jnp.float32),
            jax.ShapeDtypeStruct((1, np_tot), jnp.float32),
        ],
        compiler_params=pltpu.CompilerParams(
            dimension_semantics=("parallel",),
        ),
        interpret=interpret,
    )(xcT, w1t, b1.reshape(128, 1), w2t, b2.reshape(128, 1), w3g, b3g,
      tmat)

    y = jnp.transpose(yT[:, :n])
    log_det = ldT[0, :n]
    return y, log_det


@functools.partial(jax.jit, static_argnames=("interpret",))
def kernel(x, c, W1, b1, W2, b2, W3, b3, train, interpret=False):
    del train
    devs = jax.devices()
    nd = len(devs)
    if interpret or nd < 2 or x.shape[0] % nd != 0:
        return _coupling(x, c, W1, b1, W2, b2, W3, b3, interpret=interpret)
    mesh = jax.sharding.Mesh(np.asarray(devs), ("i",))
    p = jax.sharding.PartitionSpec
    fn = jax.experimental.shard_map.shard_map(
        _coupling, mesh=mesh,
        in_specs=(p("i"), p("i"), p(), p(), p(), p(), p(), p()),
        out_specs=(p("i"), p("i")),
        check_rep=False,
    )
    return fn(x, c, W1, b1, W2, b2, W3, b3)


# speedup vs baseline: 1.4491x; 1.4491x over previous
import functools


import numpy as np

import jax
import jax.numpy as jnp
from jax.experimental import pallas as pl
from jax.experimental.pallas import tpu as pltpu

_KNOTS = 16
_BOUND = 5.0
_LOWER = 3
_TN = 8192


def _build_tmat():
    t = np.zeros((54, 96), np.float32)
    for d in range(3):
        for k in range(16):
            t[16 * d + k, 16 * d : 16 * d + k + 1] = 1.0
        t[48 + d, 16 * d : 16 * d + 16] = 1.0
        t[51 + d, 48 + 16 * d : 48 + 16 * d + 16] = 1.0
    return t


def _spline_body(xc_ref, w1_ref, b1_ref, w2_ref, b2_ref, w3_ref,
                 b3_ref, t_ref, y_ref, ld_ref):
    xc = xc_ref[...]
    h = xc[3:11, :]
    a = jnp.dot(w1_ref[...], h, preferred_element_type=jnp.float32)
    a = jnp.maximum(a + b1_ref[...], 0.0)
    a = jnp.dot(w2_ref[...], a, preferred_element_type=jnp.float32)
    a = jnp.maximum(a + b2_ref[...], 0.0)
    p = jnp.dot(w3_ref[...], a, preferred_element_type=jnp.float32)
    p = p + b3_ref[...]

    lo = xc[0:3, :]
    tn = lo.shape[1]

    epieces = []
    for g in range(6):
        lg = p[16 * g : 16 * g + 16, :]
        epieces.append(jnp.exp(lg - jnp.max(lg, axis=0, keepdims=True)))
    e_all = jnp.concatenate(epieces, axis=0)
    dlog = p[96:144, :]
    d_all = jnp.maximum(dlog, 0.0) + jnp.log1p(jnp.exp(-jnp.abs(dlog)))

    c_mat = jnp.dot(t_ref[...], e_all,
                    preferred_element_type=jnp.float32,
                    precision=jax.lax.Precision.HIGHEST)

    oob = (lo <= -_BOUND) | (lo >= _BOUND)
    xm = jnp.where(oob, -_BOUND, lo)
    sw3 = c_mat[48:51, :]
    sh3 = c_mat[51:54, :]
    t3 = (xm + _BOUND) * sw3 * (1.0 / (2.0 * _BOUND))

    kposf = jax.lax.broadcasted_iota(jnp.int32, (_KNOTS, tn), 0).astype(
        jnp.float32)
    idx_pieces = []
    xpieces = []
    for d in range(3):
        cwe = c_mat[16 * d : 16 * d + 16, :]
        td = t3[d : d + 1, :]
        cnt = jnp.sum(jnp.where(td >= cwe, 1.0, 0.0), axis=0, keepdims=True)
        idxd = jnp.clip(cnt, 0.0, 15.0)
        mlt = jnp.where(kposf < idxd, 1.0, 0.0)
        seq = jnp.where(kposf == idxd, 1.0, 0.0)
        sm1 = jnp.where(kposf == idxd - 1.0, 1.0, 0.0)
        ew = e_all[16 * d : 16 * d + 16, :]
        eh = e_all[48 + 16 * d : 48 + 16 * d + 16, :]
        dd = d_all[16 * d : 16 * d + 16, :]
        xpieces.append([
            jnp.sum(ew * seq, axis=0, keepdims=True),
            jnp.sum(eh * seq, axis=0, keepdims=True),
            jnp.sum(ew * mlt, axis=0, keepdims=True),
            jnp.sum(eh * mlt, axis=0, keepdims=True),
            jnp.sum(dd * sm1, axis=0, keepdims=True),
            jnp.sum(dd * seq, axis=0, keepdims=True),
        ])
        idx_pieces.append(idxd)
    idx3 = jnp.concatenate(idx_pieces, axis=0)
    s_mat = jnp.concatenate(
        [xpieces[d][q] for q in range(6) for d in range(3)], axis=0)

    rw3 = (2.0 * _BOUND) / sw3
    rh3 = (2.0 * _BOUND) / sh3
    wk = s_mat[0:3, :] * rw3
    hk = s_mat[3:6, :] * rh3
    xkb = s_mat[6:9, :] * rw3 - _BOUND
    ykb = s_mat[9:12, :] * rh3 - _BOUND
    dkb = jnp.where(idx3 == 0.0, 1.0, s_mat[12:15, :])
    dk1 = jnp.where(idx3 == 15.0, 1.0, s_mat[15:18, :])

    rwk = 1.0 / wk
    sk = hk * rwk
    relx = jnp.clip((xm - xkb) * rwk, 0.0, 1.0)
    omr = 1.0 - relx
    r1 = relx * omr
    den = sk + (dk1 + dkb - 2.0 * sk) * r1
    num = hk * (sk * relx * relx + dkb * r1)
    y3 = ykb + num / den
    ld3 = (2.0 * jnp.log(sk)
           + jnp.log(dk1 * relx * relx + 2.0 * sk * r1 + dkb * omr * omr)
           - 2.0 * jnp.log(den))
    y3 = jnp.where(oob, lo, y3)
    ld3 = jnp.where(oob, 0.0, ld3)

    y_ref[0:3, :] = y3
    y_ref[3:6, :] = h[0:3, :]
    ld_ref[...] = jnp.sum(ld3, axis=0, keepdims=True)


def _coupling(x, c, W1, b1, W2, b2, W3, b3, interpret=False):
    n = x.shape[0]
    g = -(-n // _TN)
    npad = g * _TN - n

    xc = jnp.concatenate([x, c, jnp.zeros((n, 1), jnp.float32)], axis=1)
    xcT = jnp.pad(jnp.transpose(xc), ((0, 0), (0, npad)))

    w1t = jnp.pad(jnp.transpose(W1), ((0, 0), (0, 1)))
    w2t = jnp.transpose(W2)
    w3t = jnp.transpose(W3)
    z1 = jnp.zeros((1, 128), jnp.float32)
    rows = []
    brows = []
    for off, width in ((0, 16), (16, 16), (32, 15)):
        for d in range(_LOWER):
            s = 47 * d + off
            rows.append(w3t[s:s + width])
            brows.append(b3[s:s + width])
            if width == 15:
                rows.append(z1)
                brows.append(jnp.zeros((1,), jnp.float32))
    w3g = jnp.concatenate(rows, axis=0)
    b3g = jnp.concatenate(brows, axis=0).reshape(144, 1)

    tmat = jnp.asarray(_build_tmat())

    np_tot = n + npad
    yT, ldT = pl.pallas_call(
        _spline_body,
        grid=(g,),
        in_specs=[
            pl.BlockSpec((11, _TN), lambda i: (0, i)),
            pl.BlockSpec((128, 8), lambda i: (0, 0)),
            pl.BlockSpec((128, 1), lambda i: (0, 0)),
            pl.BlockSpec((128, 128), lambda i: (0, 0)),
            pl.BlockSpec((128, 1), lambda i: (0, 0)),
            pl.BlockSpec((144, 128), lambda i: (0, 0)),
            pl.BlockSpec((144, 1), lambda i: (0, 0)),
            pl.BlockSpec((54, 96), lambda i: (0, 0)),
        ],
        out_specs=[
            pl.BlockSpec((6, _TN), lambda i: (0, i)),
            pl.BlockSpec((1, _TN), lambda i: (0, i)),
        ],
        out_shape=[
            jax.ShapeDtypeStruct((6, np_tot), jnp.float32),
            jax.ShapeDtypeStruct((1, np_tot), jnp.float32),
        ],
        compiler_params=pltpu.CompilerParams(
            dimension_semantics=("parallel",),
        ),
        interpret=interpret,
    )(xcT, w1t, b1.reshape(128, 1), w2t, b2.reshape(128, 1), w3g, b3g,
      tmat)

    y = jnp.transpose(yT[:, :n])
    log_det = ldT[0, :n]
    return y, log_det


@functools.partial(jax.jit, static_argnames=("interpret",))
def kernel(x, c, W1, b1, W2, b2, W3, b3, train, interpret=False):
    del train
    return _coupling(x, c, W1, b1, W2, b2, W3, b3, interpret=interpret)


# speedup vs baseline: 1.4673x; 1.0126x over previous
import functools


import numpy as np

import jax
import jax.numpy as jnp
from jax.experimental import pallas as pl
from jax.experimental.pallas import tpu as pltpu

_KNOTS = 16
_BOUND = 5.0
_LOWER = 3
_TN = 8192


def _build_tmat():
    t = np.zeros((54, 96), np.float32)
    for d in range(3):
        for k in range(16):
            t[16 * d + k, 16 * d : 16 * d + k + 1] = 1.0
        t[48 + d, 16 * d : 16 * d + 16] = 1.0
        t[51 + d, 48 + 16 * d : 48 + 16 * d + 16] = 1.0
    return t


def _spline_body(xc_ref, w1_ref, b1_ref, w2_ref, b2_ref, w3_ref,
                 b3_ref, t_ref, y_ref, ld_ref):
    xc = xc_ref[...]
    h = xc[3:11, :]
    a = jnp.dot(w1_ref[...], h, preferred_element_type=jnp.float32)
    a = jnp.maximum(a + b1_ref[...], 0.0)
    a = jnp.dot(w2_ref[...], a, preferred_element_type=jnp.float32)
    a = jnp.maximum(a + b2_ref[...], 0.0)
    p = jnp.dot(w3_ref[...], a, preferred_element_type=jnp.float32)
    p = p + b3_ref[...]

    lo = xc[0:3, :]
    tn = lo.shape[1]

    epieces = []
    for g in range(6):
        lg = p[16 * g : 16 * g + 16, :]
        epieces.append(jnp.exp(lg - jnp.max(lg, axis=0, keepdims=True)))
    e_all = jnp.concatenate(epieces, axis=0)
    dlog = p[96:144, :]
    d_all = jnp.maximum(dlog, 0.0) + jnp.log1p(jnp.exp(-jnp.abs(dlog)))

    c_mat = jnp.dot(t_ref[...], e_all,
                    preferred_element_type=jnp.float32,
                    precision=jax.lax.Precision.HIGHEST)

    oob = (lo <= -_BOUND) | (lo >= _BOUND)
    xm = jnp.where(oob, -_BOUND, lo)
    sw3 = c_mat[48:51, :]
    sh3 = c_mat[51:54, :]
    t3 = (xm + _BOUND) * sw3 * (1.0 / (2.0 * _BOUND))

    kposf = jax.lax.broadcasted_iota(jnp.int32, (_KNOTS, tn), 0).astype(
        jnp.float32)
    idx_pieces = []
    xpieces = []
    for d in range(3):
        cwe = c_mat[16 * d : 16 * d + 16, :]
        td = t3[d : d + 1, :]
        cnt = jnp.sum(jnp.where(td >= cwe, 1.0, 0.0), axis=0, keepdims=True)
        idxd = jnp.clip(cnt, 0.0, 15.0)
        mlt = jnp.where(kposf < idxd, 1.0, 0.0)
        seq = jnp.where(kposf == idxd, 1.0, 0.0)
        sm1 = jnp.where(kposf == idxd - 1.0, 1.0, 0.0)
        ew = e_all[16 * d : 16 * d + 16, :]
        eh = e_all[48 + 16 * d : 48 + 16 * d + 16, :]
        dd = d_all[16 * d : 16 * d + 16, :]
        xpieces.append([
            jnp.sum(ew * seq, axis=0, keepdims=True),
            jnp.sum(eh * seq, axis=0, keepdims=True),
            jnp.sum(ew * mlt, axis=0, keepdims=True),
            jnp.sum(eh * mlt, axis=0, keepdims=True),
            jnp.sum(dd * sm1, axis=0, keepdims=True),
            jnp.sum(dd * seq, axis=0, keepdims=True),
        ])
        idx_pieces.append(idxd)
    idx3 = jnp.concatenate(idx_pieces, axis=0)
    s_mat = jnp.concatenate(
        [xpieces[d][q] for q in range(6) for d in range(3)], axis=0)

    rw3 = (2.0 * _BOUND) / sw3
    rh3 = (2.0 * _BOUND) / sh3
    wk = s_mat[0:3, :] * rw3
    hk = s_mat[3:6, :] * rh3
    xkb = s_mat[6:9, :] * rw3 - _BOUND
    ykb = s_mat[9:12, :] * rh3 - _BOUND
    dkb = jnp.where(idx3 == 0.0, 1.0, s_mat[12:15, :])
    dk1 = jnp.where(idx3 == 15.0, 1.0, s_mat[15:18, :])

    rwk = 1.0 / wk
    sk = hk * rwk
    relx = jnp.clip((xm - xkb) * rwk, 0.0, 1.0)
    omr = 1.0 - relx
    r1 = relx * omr
    den = sk + (dk1 + dkb - 2.0 * sk) * r1
    num = hk * (sk * relx * relx + dkb * r1)
    y3 = ykb + num / den
    ld3 = (2.0 * jnp.log(sk)
           + jnp.log(dk1 * relx * relx + 2.0 * sk * r1 + dkb * omr * omr)
           - 2.0 * jnp.log(den))
    y3 = jnp.where(oob, lo, y3)
    ld3 = jnp.where(oob, 0.0, ld3)

    y_ref[0:3, :] = y3
    y_ref[3:6, :] = h[0:3, :]
    ld_ref[...] = jnp.sum(ld3, axis=0, keepdims=True)


def _coupling(x, c, W1, b1, W2, b2, W3, b3, interpret=False):
    n = x.shape[0]
    g = -(-n // _TN)

    xc = jnp.concatenate([x, c, jnp.zeros((n, 1), jnp.float32)], axis=1)
    xcT = jnp.transpose(xc)

    w1t = jnp.pad(jnp.transpose(W1), ((0, 0), (0, 1)))
    w2t = jnp.transpose(W2)
    w3t = jnp.transpose(W3)
    z1 = jnp.zeros((1, 128), jnp.float32)
    rows = []
    brows = []
    for off, width in ((0, 16), (16, 16), (32, 15)):
        for d in range(_LOWER):
            s = 47 * d + off
            rows.append(w3t[s:s + width])
            brows.append(b3[s:s + width])
            if width == 15:
                rows.append(z1)
                brows.append(jnp.zeros((1,), jnp.float32))
    w3g = jnp.concatenate(rows, axis=0)
    b3g = jnp.concatenate(brows, axis=0).reshape(144, 1)

    tmat = jnp.asarray(_build_tmat())

    yT, ldT = pl.pallas_call(
        _spline_body,
        grid=(g,),
        in_specs=[
            pl.BlockSpec((11, _TN), lambda i: (0, i)),
            pl.BlockSpec((128, 8), lambda i: (0, 0)),
            pl.BlockSpec((128, 1), lambda i: (0, 0)),
            pl.BlockSpec((128, 128), lambda i: (0, 0)),
            pl.BlockSpec((128, 1), lambda i: (0, 0)),
            pl.BlockSpec((144, 128), lambda i: (0, 0)),
            pl.BlockSpec((144, 1), lambda i: (0, 0)),
            pl.BlockSpec((54, 96), lambda i: (0, 0)),
        ],
        out_specs=[
            pl.BlockSpec((6, _TN), lambda i: (0, i)),
            pl.BlockSpec((1, _TN), lambda i: (0, i)),
        ],
        out_shape=[
            jax.ShapeDtypeStruct((6, n), jnp.float32),
            jax.ShapeDtypeStruct((1, n), jnp.float32),
        ],
        compiler_params=pltpu.CompilerParams(
            dimension_semantics=("parallel",),
        ),
        interpret=interpret,
    )(xcT, w1t, b1.reshape(128, 1), w2t, b2.reshape(128, 1), w3g, b3g,
      tmat)

    y = jnp.transpose(yT)
    log_det = ldT[0]
    return y, log_det


@functools.partial(jax.jit, static_argnames=("interpret",))
def kernel(x, c, W1, b1, W2, b2, W3, b3, train, interpret=False):
    del train
    return _coupling(x, c, W1, b1, W2, b2, W3, b3, interpret=interpret)


# speedup vs baseline: 1.4845x; 1.0117x over previous
import functools


import numpy as np

import jax
import jax.numpy as jnp
from jax.experimental import pallas as pl
from jax.experimental.pallas import tpu as pltpu

_KNOTS = 16
_BOUND = 5.0
_LOWER = 3
_TN = 8192


def _build_tmat():
    t = np.zeros((54, 96), np.float32)
    for d in range(3):
        for k in range(16):
            t[16 * d + k, 16 * d : 16 * d + k + 1] = 1.0
        t[48 + d, 16 * d : 16 * d + 16] = 1.0
        t[51 + d, 48 + 16 * d : 48 + 16 * d + 16] = 1.0
    return t


def _spline_body(xc_ref, w1_ref, b1_ref, w2_ref, b2_ref, w3_ref,
                 b3_ref, t_ref, y_ref, ld_ref):
    xc = xc_ref[...]
    h = xc[3:11, :]
    a = jnp.dot(w1_ref[...], h, preferred_element_type=jnp.float32)
    a = jnp.maximum(a + b1_ref[...], 0.0)
    a = jnp.dot(w2_ref[...], a, preferred_element_type=jnp.float32)
    a = jnp.maximum(a + b2_ref[...], 0.0)
    p = jnp.dot(w3_ref[...], a, preferred_element_type=jnp.float32)
    p = p + b3_ref[...]

    lo = xc[0:3, :]
    tn = lo.shape[1]

    epieces = []
    for g in range(6):
        lg = p[16 * g : 16 * g + 16, :]
        epieces.append(jnp.exp(lg - jnp.max(lg, axis=0, keepdims=True)))
    e_all = jnp.concatenate(epieces, axis=0)
    dlog = p[96:144, :]
    d_all = jnp.maximum(dlog, 0.0) + jnp.log1p(jnp.exp(-jnp.abs(dlog)))

    c_mat = jnp.dot(t_ref[...], e_all,
                    preferred_element_type=jnp.float32,
                    precision=jax.lax.Precision.HIGHEST)

    oob = (lo <= -_BOUND) | (lo >= _BOUND)
    xm = jnp.where(oob, -_BOUND, lo)
    sw3 = c_mat[48:51, :]
    sh3 = c_mat[51:54, :]
    t3 = (xm + _BOUND) * sw3 * (1.0 / (2.0 * _BOUND))

    kposf = jax.lax.broadcasted_iota(jnp.int32, (_KNOTS, tn), 0).astype(
        jnp.float32)
    idx_pieces = []
    xpieces = []
    for d in range(3):
        cwe = c_mat[16 * d : 16 * d + 16, :]
        td = t3[d : d + 1, :]
        cnt = jnp.sum(jnp.where(td >= cwe, 1.0, 0.0), axis=0, keepdims=True)
        idxd = jnp.clip(cnt, 0.0, 15.0)
        mlt = kposf < idxd
        seq = kposf == idxd
        sm1 = kposf == idxd - 1.0
        ew = e_all[16 * d : 16 * d + 16, :]
        eh = e_all[48 + 16 * d : 48 + 16 * d + 16, :]
        dd = d_all[16 * d : 16 * d + 16, :]
        msum = lambda m, v: jnp.sum(jnp.where(m, v, 0.0), axis=0,
                                    keepdims=True)
        xpieces.append([
            msum(seq, ew),
            msum(seq, eh),
            msum(mlt, ew),
            msum(mlt, eh),
            msum(sm1, dd),
            msum(seq, dd),
        ])
        idx_pieces.append(idxd)
    idx3 = jnp.concatenate(idx_pieces, axis=0)
    s_mat = jnp.concatenate(
        [xpieces[d][q] for q in range(6) for d in range(3)], axis=0)

    rw3 = (2.0 * _BOUND) / sw3
    rh3 = (2.0 * _BOUND) / sh3
    wk = s_mat[0:3, :] * rw3
    hk = s_mat[3:6, :] * rh3
    xkb = s_mat[6:9, :] * rw3 - _BOUND
    ykb = s_mat[9:12, :] * rh3 - _BOUND
    dkb = jnp.where(idx3 == 0.0, 1.0, s_mat[12:15, :])
    dk1 = jnp.where(idx3 == 15.0, 1.0, s_mat[15:18, :])

    rwk = 1.0 / wk
    sk = hk * rwk
    relx = jnp.clip((xm - xkb) * rwk, 0.0, 1.0)
    omr = 1.0 - relx
    r1 = relx * omr
    den = sk + (dk1 + dkb - 2.0 * sk) * r1
    num = hk * (sk * relx * relx + dkb * r1)
    y3 = ykb + num / den
    ld3 = (2.0 * jnp.log(sk)
           + jnp.log(dk1 * relx * relx + 2.0 * sk * r1 + dkb * omr * omr)
           - 2.0 * jnp.log(den))
    y3 = jnp.where(oob, lo, y3)
    ld3 = jnp.where(oob, 0.0, ld3)

    y_ref[0:3, :] = y3
    y_ref[3:6, :] = h[0:3, :]
    ld_ref[...] = jnp.sum(ld3, axis=0, keepdims=True)


def _coupling(x, c, W1, b1, W2, b2, W3, b3, interpret=False):
    n = x.shape[0]
    g = -(-n // _TN)

    xc = jnp.concatenate([x, c, jnp.zeros((n, 1), jnp.float32)], axis=1)
    xcT = jnp.transpose(xc)

    w1t = jnp.pad(jnp.transpose(W1), ((0, 0), (0, 1)))
    w2t = jnp.transpose(W2)
    w3t = jnp.transpose(W3)
    z1 = jnp.zeros((1, 128), jnp.float32)
    rows = []
    brows = []
    for off, width in ((0, 16), (16, 16), (32, 15)):
        for d in range(_LOWER):
            s = 47 * d + off
            rows.append(w3t[s:s + width])
            brows.append(b3[s:s + width])
            if width == 15:
                rows.append(z1)
                brows.append(jnp.zeros((1,), jnp.float32))
    w3g = jnp.concatenate(rows, axis=0)
    b3g = jnp.concatenate(brows, axis=0).reshape(144, 1)

    tmat = jnp.asarray(_build_tmat())

    yT, ldT = pl.pallas_call(
        _spline_body,
        grid=(g,),
        in_specs=[
            pl.BlockSpec((11, _TN), lambda i: (0, i)),
            pl.BlockSpec((128, 8), lambda i: (0, 0)),
            pl.BlockSpec((128, 1), lambda i: (0, 0)),
            pl.BlockSpec((128, 128), lambda i: (0, 0)),
            pl.BlockSpec((128, 1), lambda i: (0, 0)),
            pl.BlockSpec((144, 128), lambda i: (0, 0)),
            pl.BlockSpec((144, 1), lambda i: (0, 0)),
            pl.BlockSpec((54, 96), lambda i: (0, 0)),
        ],
        out_specs=[
            pl.BlockSpec((6, _TN), lambda i: (0, i)),
            pl.BlockSpec((1, _TN), lambda i: (0, i)),
        ],
        out_shape=[
            jax.ShapeDtypeStruct((6, n), jnp.float32),
            jax.ShapeDtypeStruct((1, n), jnp.float32),
        ],
        compiler_params=pltpu.CompilerParams(
            dimension_semantics=("parallel",),
        ),
        interpret=interpret,
    )(xcT, w1t, b1.reshape(128, 1), w2t, b2.reshape(128, 1), w3g, b3g,
      tmat)

    y = jnp.transpose(yT)
    log_det = ldT[0]
    return y, log_det


@functools.partial(jax.jit, static_argnames=("interpret",))
def kernel(x, c, W1, b1, W2, b2, W3, b3, train, interpret=False):
    del train
    return _coupling(x, c, W1, b1, W2, b2, W3, b3, interpret=interpret)
